# baseline (speedup 1.0000x reference)
import functools

import jax
import jax.numpy as jnp
from jax import lax
from jax.experimental import pallas as pl
from jax.experimental.pallas import tpu as pltpu
from jax.experimental.pallas import tpu_sc as plsc

LANES = 16


def _pos_embed_body(nb, n_chan, h, w, d, cpw, nc,
                    g_hbm, g16_hbm, out_hbm, g_vmem, g16_vmem,
                    vol0, vol1, sem0, sem1):
    q = w * d
    vregs_per_wblock = d // LANES

    wid = lax.axis_index("s") * nc + lax.axis_index("c")
    base = wid * cpw
    vols = (vol0, vol1)
    sems = (sem0, sem1)

    for i in range(cpw):
        p = i % 2
        vol = vols[p]
        sem = sems[p]
        c = base + i

        if i >= 2:
            for _ in range(nb):
                pltpu.make_async_copy(vol, out_hbm.at[0, 0], sem).wait()

        pltpu.sync_copy(g_hbm.at[c], g_vmem)
        pltpu.sync_copy(g16_hbm.at[c], g16_vmem)

        seg = c // (n_chan // 3)

        @pl.when(seg == 0)
        def _():
            def per_w(wi, carry):
                v = g16_vmem[pl.ds(wi * LANES, LANES)]
                for hi in range(h):
                    for j in range(vregs_per_wblock):
                        vol[pl.ds(hi * q + wi * d + j * LANES, LANES)] = v
                return carry
            lax.fori_loop(0, w, per_w, 0)

        @pl.when(seg == 1)
        def _():
            def per_h(hi, carry):
                v = g16_vmem[pl.ds(hi * LANES, LANES)]
                for j in range(q // LANES):
                    vol[pl.ds(hi * q + j * LANES, LANES)] = v
                return carry
            lax.fori_loop(0, h, per_h, 0)

        @pl.when(seg == 2)
        def _():
            gv = [g_vmem[pl.ds(j * LANES, LANES)] for j in range(d // LANES)]
            def per_h(hi, carry):
                for j in range(q // LANES):
                    vol[pl.ds(hi * q + j * LANES, LANES)] = gv[j % len(gv)]
                return carry
            lax.fori_loop(0, h, per_h, 0)

        for b in range(nb):
            pltpu.async_copy(vol, out_hbm.at[b, c], sem)

    for i in range(min(2, cpw)):
        for _ in range(nb):
            pltpu.make_async_copy(vols[i], out_hbm.at[0, 0], sems[i]).wait()


def kernel(tensor_list, row_embed_weight, col_embed_weight, depth_embed_weight):
    x = tensor_list
    h, w, d = x.shape[-3], x.shape[-2], x.shape[-1]
    nb = x.shape[0]
    f = row_embed_weight.shape[-1]
    n_chan = 3 * f
    vol_len = h * w * d

    g = jnp.concatenate(
        [col_embed_weight[:w].T, row_embed_weight[:h].T, depth_embed_weight[:d].T],
        axis=0,
    )
    g16 = jnp.repeat(g, LANES, axis=1)

    info = plsc.get_sparse_core_info()
    nc, ns = info.num_cores, info.num_subcores
    nw = nc * ns
    cpw = n_chan // nw

    run = pl.kernel(
        functools.partial(_pos_embed_body, nb, n_chan, h, w, d, cpw, nc),
        mesh=plsc.VectorSubcoreMesh(core_axis_name="c", subcore_axis_name="s"),
        out_type=jax.ShapeDtypeStruct((nb, n_chan, vol_len), jnp.float32),
        scratch_types=[
            pltpu.VMEM((w,), jnp.float32),
            pltpu.VMEM((w * LANES,), jnp.float32),
            pltpu.VMEM((vol_len,), jnp.float32),
            pltpu.VMEM((vol_len,), jnp.float32),
            pltpu.SemaphoreType.DMA,
            pltpu.SemaphoreType.DMA,
        ],
    )
    out = run(g, g16)
    return out.reshape(nb, n_chan, h, w, d)
